# baseline (speedup 1.0000x reference)
import functools

import jax
import jax.numpy as jnp
from jax import lax
from jax.experimental import pallas as pl
from jax.experimental.pallas import tpu as pltpu
from jax.experimental.pallas import tpu_sc as plsc

D = 100
DP = 128
T, S, Q = 64, 32, 32
NTOK = T * S
NE = 8 * NTOK
NR = 4 * NTOK
NC, NS = 2, 16
NW = NC * NS
CH = 128
E_PER = NE // NW
R_PER = NR // NW
E_CH = E_PER // CH
R_CH = R_PER // CH


def _make_gather_kernel(n_ch):
    per = n_ch * CH

    def k(tab_hbm, idx_hbm, out_hbm, idx_v, rows_v, sem):
        wid = lax.axis_index("s") * NC + lax.axis_index("c")
        pltpu.sync_copy(idx_hbm.at[wid], idx_v)
        copies = [
            pltpu.async_copy(tab_hbm.at[idx_v.at[j]],
                             rows_v.at[pl.ds(j * CH, CH)], sem)
            for j in range(n_ch)
        ]
        for c in copies:
            c.wait()
        pltpu.sync_copy(rows_v, out_hbm.at[pl.ds(wid * per, per)])

    return k


def _sc_gather(table, idx, n_ch):
    per = n_ch * CH
    mesh = plsc.VectorSubcoreMesh(core_axis_name="c", subcore_axis_name="s")
    run = functools.partial(
        pl.kernel,
        mesh=mesh,
        out_type=jax.ShapeDtypeStruct((NW * per, DP), jnp.float32),
        scratch_types=[
            pltpu.VMEM((n_ch, CH), jnp.int32),
            pltpu.VMEM((per, DP), jnp.float32),
            pltpu.SemaphoreType.DMA,
        ],
    )(_make_gather_kernel(n_ch))
    return run(table, idx.reshape(NW, n_ch, CH))


TP_BLK = 12800
TP_GRID = (100000 + TP_BLK - 1) // TP_BLK


def _tp_kernel(x_ref, e_ref, o_ref):
    o_ref[...] = jax.lax.dot_general(
        x_ref[...], e_ref[...], (((0,), (0,)), ((), ())),
        preferred_element_type=jnp.float32)


def _transpose_pad(tab_t, eye_p):
    return pl.pallas_call(
        _tp_kernel,
        grid=(TP_GRID,),
        in_specs=[pl.BlockSpec((D, TP_BLK), lambda i: (0, i)),
                  pl.BlockSpec((D, DP), lambda i: (0, 0))],
        out_specs=pl.BlockSpec((TP_BLK, DP), lambda i: (i, 0)),
        out_shape=jax.ShapeDtypeStruct((100000, DP), jnp.float32),
    )(tab_t, eye_p)


def _dense_kernel(e_ref, r_ref, m_ref, wagg_ref, bagg_ref, w1_ref, b1_ref,
                  w2_ref, b2_ref, w3_ref, b3_ref, wmu_ref, bmu_ref,
                  wsig_ref, bsig_ref, wd_ref, bd_ref, eps_ref,
                  sp_out, sn_out, kld_out):
    f32 = jnp.float32

    def seg_e(i):
        return e_ref[pl.ds(i * NTOK, NTOK), pl.ds(0, D)]

    def seg_r(i):
        return r_ref[pl.ds(i * NTOK, NTOK), pl.ds(0, D)]

    def mask(i):
        return m_ref[:, i:i + 1]

    sp_t = seg_e(1)
    sp_r = seg_r(0)
    wagg_t = wagg_ref[pl.ds(0, D), :]
    wagg_r = wagg_ref[pl.ds(D, D), :]
    nbr = jnp.maximum(
        jnp.dot(sp_t, wagg_t, preferred_element_type=f32)
        + jnp.dot(sp_r, wagg_r, preferred_element_type=f32)
        + bagg_ref[0, :], 0.0)
    u = jnp.sum(nbr.reshape(T, S, D), axis=1) * (1.0 / S)
    u_rep = jnp.repeat(u, S, axis=0)

    def blend(x, m):
        return x + m * (u_rep - x)

    bf = jnp.bfloat16
    w1_h = w1_ref[pl.ds(0, D), :].astype(bf)
    w1_r = w1_ref[pl.ds(D, D), :].astype(bf)
    w1_t = w1_ref[pl.ds(2 * D, D), :].astype(bf)
    w2_bf = w2_ref[...].astype(bf)
    w1_lab = w1_ref[pl.ds(3 * D, 1), :]
    labels = (1.0, 0.0, 1.0, 0.0)
    pooled = []
    for s in range(4):
        h_emb = blend(seg_e(2 * s), mask(2 * s))
        t_emb = blend(seg_e(2 * s + 1), mask(2 * s + 1))
        r_emb = seg_r(s)
        h1 = jnp.maximum(
            jnp.dot(h_emb.astype(bf), w1_h, preferred_element_type=f32)
            + jnp.dot(r_emb.astype(bf), w1_r, preferred_element_type=f32)
            + jnp.dot(t_emb.astype(bf), w1_t, preferred_element_type=f32)
            + (b1_ref[0, :] + labels[s] * w1_lab[0, :]), 0.0)
        h2 = jnp.maximum(
            jnp.dot(h1.astype(bf), w2_bf, preferred_element_type=f32)
            + b2_ref[0, :], 0.0)
        pooled.append(jnp.sum(h2.reshape(T, S, 200), axis=1))

    c_h = (pooled[0] + pooled[1]) * (1.0 / (2 * S))
    t_h = (pooled[0] + pooled[1] + pooled[2] + pooled[3]) * (1.0 / (4 * S))
    c = jnp.dot(c_h, w3_ref[...], preferred_element_type=f32) + b3_ref[0, :]
    tg = jnp.dot(t_h, w3_ref[...], preferred_element_type=f32) + b3_ref[0, :]

    mu_c = jnp.dot(c, wmu_ref[...], preferred_element_type=f32) + bmu_ref[0, :]
    mu_t = jnp.dot(tg, wmu_ref[...], preferred_element_type=f32) + bmu_ref[0, :]
    sg_c = 0.1 + 0.9 * jax.nn.sigmoid(
        jnp.dot(c, wsig_ref[...], preferred_element_type=f32) + bsig_ref[0, :])
    sg_t = 0.1 + 0.9 * jax.nn.sigmoid(
        jnp.dot(tg, wsig_ref[...], preferred_element_type=f32) + bsig_ref[0, :])

    kld = jnp.sum(
        jnp.log(sg_c / sg_t)
        + (sg_t * sg_t + (mu_t - mu_c) ** 2) / (2.0 * sg_c * sg_c) - 0.5,
        axis=-1, keepdims=True)
    kld_out[...] = kld

    z = mu_t + sg_t * eps_ref[...]
    z_rep = jnp.repeat(z, S, axis=0)
    wd_e = wd_ref[pl.ds(0, D), :]
    wd_z = wd_ref[pl.ds(D, D), :]
    z_dec = jnp.dot(z_rep, wd_z, preferred_element_type=f32) + bd_ref[0, :]

    for s, out in ((2, sp_out), (3, sn_out)):
        h_emb = blend(seg_e(2 * s), mask(2 * s))
        t_emb = blend(seg_e(2 * s + 1), mask(2 * s + 1))
        r_emb = seg_r(s)
        h_dec = jnp.dot(h_emb, wd_e, preferred_element_type=f32) + z_dec
        t_dec = jnp.dot(t_emb, wd_e, preferred_element_type=f32) + z_dec
        prod = h_dec * r_emb * t_dec
        out[...] = jnp.sum(prod.reshape(T, Q, D), axis=-1)


def _dense(e_rows, r_rows, masks, W_agg, b_agg, W1, b1, W2, b2, W3, b3,
           Wmu, bmu, Wsig, bsig, Wd, bd, eps, interpret=False):
    return pl.pallas_call(
        _dense_kernel,
        out_shape=(jax.ShapeDtypeStruct((T, Q), jnp.float32),
                   jax.ShapeDtypeStruct((T, Q), jnp.float32),
                   jax.ShapeDtypeStruct((T, 1), jnp.float32)),
        interpret=interpret,
    )(e_rows, r_rows, masks, W_agg, b_agg.reshape(1, D), W1,
      b1.reshape(1, 500), W2, b2.reshape(1, 200), W3, b3.reshape(1, D),
      Wmu, bmu.reshape(1, D), Wsig, bsig.reshape(1, D), Wd,
      bd.reshape(1, D), eps)


def kernel(entity_table, relation_table, W_agg, b_agg, W1, b1, W2, b2, W3,
           b3, Wmu, bmu, Wsig, bsig, Wd, bd, unseen_entities, support_pos,
           support_neg, query_pos, query_neg):
    sets = [support_pos.reshape(NTOK, 3), support_neg.reshape(NTOK, 3),
            query_pos.reshape(NTOK, 3), query_neg.reshape(NTOK, 3)]
    ent_idx = jnp.concatenate(
        [c for tr in sets for c in (tr[:, 0], tr[:, 2])])
    rel_idx = jnp.concatenate([tr[:, 1] for tr in sets])
    un_rep = jnp.repeat(unseen_entities, S)
    masks = jnp.stack(
        [c for tr in sets for c in (tr[:, 0] == un_rep, tr[:, 2] == un_rep)],
        axis=1).astype(jnp.float32)

    eye_p = jnp.eye(D, DP, dtype=jnp.float32)
    etab = _transpose_pad(entity_table.T, eye_p)
    e_rows = _sc_gather(etab, ent_idx, E_CH)
    rtab = _transpose_pad(relation_table.T, eye_p)
    r_rows = _sc_gather(rtab, rel_idx, R_CH)

    eps = jax.random.normal(jax.random.key(42), (T, D), jnp.float32)
    sp, sn, kld = _dense(e_rows, r_rows, masks, W_agg, b_agg, W1, b1, W2,
                         b2, W3, b3, Wmu, bmu, Wsig, bsig, Wd, bd, eps)
    return jnp.concatenate([sp, sn, kld], axis=1)


# speedup vs baseline: 1.0142x; 1.0142x over previous
import functools

import jax
import jax.numpy as jnp
from jax import lax
from jax.experimental import pallas as pl
from jax.experimental.pallas import tpu as pltpu
from jax.experimental.pallas import tpu_sc as plsc

D = 100
DP = 128
T, S, Q = 64, 32, 32
NTOK = T * S
NE = 8 * NTOK
NR = 4 * NTOK
NC, NS = 2, 16
NW = NC * NS
CH = 128
E_PER = NE // NW
R_PER = NR // NW
E_CH = E_PER // CH
R_CH = R_PER // CH


def _make_gather_kernel(n_ch):
    per = n_ch * CH

    def k(tab_hbm, idx_hbm, out_hbm, idx_v, rows_v, sem):
        wid = lax.axis_index("s") * NC + lax.axis_index("c")
        pltpu.sync_copy(idx_hbm.at[wid], idx_v)
        copies = [
            pltpu.async_copy(tab_hbm.at[idx_v.at[j]],
                             rows_v.at[pl.ds(j * CH, CH)], sem)
            for j in range(n_ch)
        ]
        for c in copies:
            c.wait()
        pltpu.sync_copy(rows_v, out_hbm.at[pl.ds(wid * per, per)])

    return k


def _sc_gather(table, idx, n_ch):
    per = n_ch * CH
    mesh = plsc.VectorSubcoreMesh(core_axis_name="c", subcore_axis_name="s")
    run = functools.partial(
        pl.kernel,
        mesh=mesh,
        out_type=jax.ShapeDtypeStruct((NW * per, DP), jnp.float32),
        scratch_types=[
            pltpu.VMEM((n_ch, CH), jnp.int32),
            pltpu.VMEM((per, DP), jnp.float32),
            pltpu.SemaphoreType.DMA,
        ],
    )(_make_gather_kernel(n_ch))
    return run(table, idx.reshape(NW, n_ch, CH))


TP_BLK = 12800
TP_GRID = (100000 + TP_BLK - 1) // TP_BLK


def _tp_kernel(x_ref, e_ref, o_ref):
    o_ref[...] = jax.lax.dot_general(
        x_ref[...], e_ref[...], (((0,), (0,)), ((), ())),
        preferred_element_type=jnp.float32)


def _transpose_pad(tab_t, eye_p):
    return pl.pallas_call(
        _tp_kernel,
        grid=(TP_GRID,),
        in_specs=[pl.BlockSpec((D, TP_BLK), lambda i: (0, i)),
                  pl.BlockSpec((D, DP), lambda i: (0, 0))],
        out_specs=pl.BlockSpec((TP_BLK, DP), lambda i: (i, 0)),
        out_shape=jax.ShapeDtypeStruct((100000, DP), jnp.float32),
    )(tab_t, eye_p)


def _dense_kernel(e_ref, r_ref, m_ref, wagg_ref, bagg_ref, w1_ref, b1_ref,
                  w2_ref, b2_ref, w3_ref, b3_ref, wmu_ref, bmu_ref,
                  wsig_ref, bsig_ref, wd_ref, bd_ref, eps_ref, out_ref):
    f32 = jnp.float32

    def seg_e(i):
        return e_ref[pl.ds(i * NTOK, NTOK), pl.ds(0, D)]

    def seg_r(i):
        return r_ref[pl.ds(i * NTOK, NTOK), pl.ds(0, D)]

    def mask(i):
        return m_ref[:, i:i + 1]

    sp_t = seg_e(1)
    sp_r = seg_r(0)
    wagg_t = wagg_ref[pl.ds(0, D), :]
    wagg_r = wagg_ref[pl.ds(D, D), :]
    nbr = jnp.maximum(
        jnp.dot(sp_t, wagg_t, preferred_element_type=f32)
        + jnp.dot(sp_r, wagg_r, preferred_element_type=f32)
        + bagg_ref[0, :], 0.0)
    u = jnp.sum(nbr.reshape(T, S, D), axis=1) * (1.0 / S)
    u_rep = jnp.repeat(u, S, axis=0)

    def blend(x, m):
        return x + m * (u_rep - x)

    bf = jnp.bfloat16
    w1_h = w1_ref[pl.ds(0, D), :].astype(bf)
    w1_r = w1_ref[pl.ds(D, D), :].astype(bf)
    w1_t = w1_ref[pl.ds(2 * D, D), :].astype(bf)
    w2_bf = w2_ref[...].astype(bf)
    w1_lab = w1_ref[pl.ds(3 * D, 1), :]
    labels = (1.0, 0.0, 1.0, 0.0)
    pooled = []
    for s in range(4):
        h_emb = blend(seg_e(2 * s), mask(2 * s))
        t_emb = blend(seg_e(2 * s + 1), mask(2 * s + 1))
        r_emb = seg_r(s)
        h1 = jnp.maximum(
            jnp.dot(h_emb.astype(bf), w1_h, preferred_element_type=f32)
            + jnp.dot(r_emb.astype(bf), w1_r, preferred_element_type=f32)
            + jnp.dot(t_emb.astype(bf), w1_t, preferred_element_type=f32)
            + (b1_ref[0, :] + labels[s] * w1_lab[0, :]), 0.0)
        h2 = jnp.maximum(
            jnp.dot(h1.astype(bf), w2_bf, preferred_element_type=f32)
            + b2_ref[0, :], 0.0)
        pooled.append(jnp.sum(h2.reshape(T, S, 200), axis=1))

    c_h = (pooled[0] + pooled[1]) * (1.0 / (2 * S))
    t_h = (pooled[0] + pooled[1] + pooled[2] + pooled[3]) * (1.0 / (4 * S))
    c = jnp.dot(c_h, w3_ref[...], preferred_element_type=f32) + b3_ref[0, :]
    tg = jnp.dot(t_h, w3_ref[...], preferred_element_type=f32) + b3_ref[0, :]

    mu_c = jnp.dot(c, wmu_ref[...], preferred_element_type=f32) + bmu_ref[0, :]
    mu_t = jnp.dot(tg, wmu_ref[...], preferred_element_type=f32) + bmu_ref[0, :]
    sg_c = 0.1 + 0.9 * jax.nn.sigmoid(
        jnp.dot(c, wsig_ref[...], preferred_element_type=f32) + bsig_ref[0, :])
    sg_t = 0.1 + 0.9 * jax.nn.sigmoid(
        jnp.dot(tg, wsig_ref[...], preferred_element_type=f32) + bsig_ref[0, :])

    kld = jnp.sum(
        jnp.log(sg_c / sg_t)
        + (sg_t * sg_t + (mu_t - mu_c) ** 2) / (2.0 * sg_c * sg_c) - 0.5,
        axis=-1, keepdims=True)
    out_ref[:, 64:65] = kld

    z = mu_t + sg_t * eps_ref[...]
    z_rep = jnp.repeat(z, S, axis=0)
    wd_e = wd_ref[pl.ds(0, D), :]
    wd_z = wd_ref[pl.ds(D, D), :]
    z_dec = jnp.dot(z_rep, wd_z, preferred_element_type=f32) + bd_ref[0, :]

    for s, col in ((2, 0), (3, Q)):
        h_emb = blend(seg_e(2 * s), mask(2 * s))
        t_emb = blend(seg_e(2 * s + 1), mask(2 * s + 1))
        r_emb = seg_r(s)
        h_dec = jnp.dot(h_emb, wd_e, preferred_element_type=f32) + z_dec
        t_dec = jnp.dot(t_emb, wd_e, preferred_element_type=f32) + z_dec
        prod = h_dec * r_emb * t_dec
        out_ref[:, col:col + Q] = jnp.sum(prod.reshape(T, Q, D), axis=-1)


def _dense(e_rows, r_rows, masks, W_agg, b_agg, W1, b1, W2, b2, W3, b3,
           Wmu, bmu, Wsig, bsig, Wd, bd, eps, interpret=False):
    return pl.pallas_call(
        _dense_kernel,
        out_shape=jax.ShapeDtypeStruct((T, 65), jnp.float32),
        interpret=interpret,
    )(e_rows, r_rows, masks, W_agg, b_agg.reshape(1, D), W1,
      b1.reshape(1, 500), W2, b2.reshape(1, 200), W3, b3.reshape(1, D),
      Wmu, bmu.reshape(1, D), Wsig, bsig.reshape(1, D), Wd,
      bd.reshape(1, D), eps)


def kernel(entity_table, relation_table, W_agg, b_agg, W1, b1, W2, b2, W3,
           b3, Wmu, bmu, Wsig, bsig, Wd, bd, unseen_entities, support_pos,
           support_neg, query_pos, query_neg):
    sets = [support_pos.reshape(NTOK, 3), support_neg.reshape(NTOK, 3),
            query_pos.reshape(NTOK, 3), query_neg.reshape(NTOK, 3)]
    ent_idx = jnp.concatenate(
        [c for tr in sets for c in (tr[:, 0], tr[:, 2])])
    rel_idx = jnp.concatenate([tr[:, 1] for tr in sets])
    un_rep = jnp.repeat(unseen_entities, S)
    masks = jnp.stack(
        [c for tr in sets for c in (tr[:, 0] == un_rep, tr[:, 2] == un_rep)],
        axis=1).astype(jnp.float32)

    eye_p = jnp.eye(D, DP, dtype=jnp.float32)
    etab = _transpose_pad(entity_table.T, eye_p)
    e_rows = _sc_gather(etab, ent_idx, E_CH)
    rtab = _transpose_pad(relation_table.T, eye_p)
    r_rows = _sc_gather(rtab, rel_idx, R_CH)

    eps = jax.random.normal(jax.random.key(42), (T, D), jnp.float32)
    return _dense(e_rows, r_rows, masks, W_agg, b_agg, W1, b1, W2,
                  b2, W3, b3, Wmu, bmu, Wsig, bsig, Wd, bd, eps)


# speedup vs baseline: 1.0290x; 1.0145x over previous
import functools

import jax
import jax.numpy as jnp
from jax import lax
from jax.experimental import pallas as pl
from jax.experimental.pallas import tpu as pltpu
from jax.experimental.pallas import tpu_sc as plsc

D = 100
DP = 128
T, S, Q = 64, 32, 32
NTOK = T * S
NE = 8 * NTOK
NR = 4 * NTOK
NC, NS = 2, 16
NW = NC * NS
CH = 128
E_PER = NE // NW
R_PER = NR // NW
E_CH = E_PER // CH
R_CH = R_PER // CH


def _make_gather_kernel(n_ch):
    per = n_ch * CH

    def k(tab_hbm, idx_hbm, out_hbm, idx_v, rows_v, sem):
        wid = lax.axis_index("s") * NC + lax.axis_index("c")
        pltpu.sync_copy(idx_hbm.at[wid], idx_v)
        copies = [
            pltpu.async_copy(tab_hbm.at[idx_v.at[j]],
                             rows_v.at[pl.ds(j * CH, CH)], sem)
            for j in range(n_ch)
        ]
        for c in copies:
            c.wait()
        pltpu.sync_copy(rows_v, out_hbm.at[pl.ds(wid * per, per)])

    return k


def _sc_gather(table, idx, n_ch):
    per = n_ch * CH
    mesh = plsc.VectorSubcoreMesh(core_axis_name="c", subcore_axis_name="s")
    run = functools.partial(
        pl.kernel,
        mesh=mesh,
        out_type=jax.ShapeDtypeStruct((NW * per, DP), jnp.float32),
        scratch_types=[
            pltpu.VMEM((n_ch, CH), jnp.int32),
            pltpu.VMEM((per, DP), jnp.float32),
            pltpu.SemaphoreType.DMA,
        ],
    )(_make_gather_kernel(n_ch))
    return run(table, idx.reshape(NW, n_ch, CH))


TP_BLK = 25600
TP_GRID = (100000 + TP_BLK - 1) // TP_BLK


def _tp_kernel(x_ref, e_ref, o_ref):
    o_ref[...] = jax.lax.dot_general(
        x_ref[...], e_ref[...], (((0,), (0,)), ((), ())),
        preferred_element_type=jnp.float32)


def _transpose_pad(tab_t, eye_p):
    return pl.pallas_call(
        _tp_kernel,
        grid=(TP_GRID,),
        in_specs=[pl.BlockSpec((D, TP_BLK), lambda i: (0, i)),
                  pl.BlockSpec((D, DP), lambda i: (0, 0))],
        out_specs=pl.BlockSpec((TP_BLK, DP), lambda i: (i, 0)),
        out_shape=jax.ShapeDtypeStruct((100000, DP), jnp.float32),
        compiler_params=pltpu.CompilerParams(vmem_limit_bytes=56 * 2**20),
    )(tab_t, eye_p)


def _dense_kernel(e_ref, r_ref, m_ref, wagg_ref, bagg_ref, w1_ref, b1_ref,
                  w2_ref, b2_ref, w3_ref, b3_ref, wmu_ref, bmu_ref,
                  wsig_ref, bsig_ref, wd_ref, bd_ref, eps_ref, out_ref):
    f32 = jnp.float32

    def seg_e(i):
        return e_ref[pl.ds(i * NTOK, NTOK), pl.ds(0, D)]

    def seg_r(i):
        return r_ref[pl.ds(i * NTOK, NTOK), pl.ds(0, D)]

    def mask(i):
        return m_ref[:, i:i + 1]

    sp_t = seg_e(1)
    sp_r = seg_r(0)
    wagg_t = wagg_ref[pl.ds(0, D), :]
    wagg_r = wagg_ref[pl.ds(D, D), :]
    nbr = jnp.maximum(
        jnp.dot(sp_t, wagg_t, preferred_element_type=f32)
        + jnp.dot(sp_r, wagg_r, preferred_element_type=f32)
        + bagg_ref[0, :], 0.0)
    u = jnp.sum(nbr.reshape(T, S, D), axis=1) * (1.0 / S)
    u_rep = jnp.repeat(u, S, axis=0)

    def blend(x, m):
        return x + m * (u_rep - x)

    bf = jnp.bfloat16
    w1_h = w1_ref[pl.ds(0, D), :].astype(bf)
    w1_r = w1_ref[pl.ds(D, D), :].astype(bf)
    w1_t = w1_ref[pl.ds(2 * D, D), :].astype(bf)
    w2_bf = w2_ref[...].astype(bf)
    w1_lab = w1_ref[pl.ds(3 * D, 1), :]
    labels = (1.0, 0.0, 1.0, 0.0)
    pooled = []
    for s in range(4):
        h_emb = blend(seg_e(2 * s), mask(2 * s))
        t_emb = blend(seg_e(2 * s + 1), mask(2 * s + 1))
        r_emb = seg_r(s)
        h1 = jnp.maximum(
            jnp.dot(h_emb.astype(bf), w1_h, preferred_element_type=f32)
            + jnp.dot(r_emb.astype(bf), w1_r, preferred_element_type=f32)
            + jnp.dot(t_emb.astype(bf), w1_t, preferred_element_type=f32)
            + (b1_ref[0, :] + labels[s] * w1_lab[0, :]), 0.0)
        h2 = jnp.maximum(
            jnp.dot(h1.astype(bf), w2_bf, preferred_element_type=f32)
            + b2_ref[0, :], 0.0)
        pooled.append(jnp.sum(h2.reshape(T, S, 200), axis=1))

    c_h = (pooled[0] + pooled[1]) * (1.0 / (2 * S))
    t_h = (pooled[0] + pooled[1] + pooled[2] + pooled[3]) * (1.0 / (4 * S))
    c = jnp.dot(c_h, w3_ref[...], preferred_element_type=f32) + b3_ref[0, :]
    tg = jnp.dot(t_h, w3_ref[...], preferred_element_type=f32) + b3_ref[0, :]

    mu_c = jnp.dot(c, wmu_ref[...], preferred_element_type=f32) + bmu_ref[0, :]
    mu_t = jnp.dot(tg, wmu_ref[...], preferred_element_type=f32) + bmu_ref[0, :]
    sg_c = 0.1 + 0.9 * jax.nn.sigmoid(
        jnp.dot(c, wsig_ref[...], preferred_element_type=f32) + bsig_ref[0, :])
    sg_t = 0.1 + 0.9 * jax.nn.sigmoid(
        jnp.dot(tg, wsig_ref[...], preferred_element_type=f32) + bsig_ref[0, :])

    kld = jnp.sum(
        jnp.log(sg_c / sg_t)
        + (sg_t * sg_t + (mu_t - mu_c) ** 2) / (2.0 * sg_c * sg_c) - 0.5,
        axis=-1, keepdims=True)
    out_ref[:, 64:65] = kld

    z = mu_t + sg_t * eps_ref[...]
    z_rep = jnp.repeat(z, S, axis=0)
    wd_e = wd_ref[pl.ds(0, D), :]
    wd_z = wd_ref[pl.ds(D, D), :]
    z_dec = jnp.dot(z_rep, wd_z, preferred_element_type=f32) + bd_ref[0, :]

    for s, col in ((2, 0), (3, Q)):
        h_emb = blend(seg_e(2 * s), mask(2 * s))
        t_emb = blend(seg_e(2 * s + 1), mask(2 * s + 1))
        r_emb = seg_r(s)
        h_dec = jnp.dot(h_emb, wd_e, preferred_element_type=f32) + z_dec
        t_dec = jnp.dot(t_emb, wd_e, preferred_element_type=f32) + z_dec
        prod = h_dec * r_emb * t_dec
        out_ref[:, col:col + Q] = jnp.sum(prod.reshape(T, Q, D), axis=-1)


def _dense(e_rows, r_rows, masks, W_agg, b_agg, W1, b1, W2, b2, W3, b3,
           Wmu, bmu, Wsig, bsig, Wd, bd, eps, interpret=False):
    return pl.pallas_call(
        _dense_kernel,
        out_shape=jax.ShapeDtypeStruct((T, 65), jnp.float32),
        interpret=interpret,
    )(e_rows, r_rows, masks, W_agg, b_agg.reshape(1, D), W1,
      b1.reshape(1, 500), W2, b2.reshape(1, 200), W3, b3.reshape(1, D),
      Wmu, bmu.reshape(1, D), Wsig, bsig.reshape(1, D), Wd,
      bd.reshape(1, D), eps)


def kernel(entity_table, relation_table, W_agg, b_agg, W1, b1, W2, b2, W3,
           b3, Wmu, bmu, Wsig, bsig, Wd, bd, unseen_entities, support_pos,
           support_neg, query_pos, query_neg):
    sets = [support_pos.reshape(NTOK, 3), support_neg.reshape(NTOK, 3),
            query_pos.reshape(NTOK, 3), query_neg.reshape(NTOK, 3)]
    ent_idx = jnp.concatenate(
        [c for tr in sets for c in (tr[:, 0], tr[:, 2])])
    rel_idx = jnp.concatenate([tr[:, 1] for tr in sets])
    un_rep = jnp.repeat(unseen_entities, S)
    masks = jnp.stack(
        [c for tr in sets for c in (tr[:, 0] == un_rep, tr[:, 2] == un_rep)],
        axis=1).astype(jnp.float32)

    eye_p = jnp.eye(D, DP, dtype=jnp.float32)
    etab = _transpose_pad(entity_table.T, eye_p)
    e_rows = _sc_gather(etab, ent_idx, E_CH)
    rtab = _transpose_pad(relation_table.T, eye_p)
    r_rows = _sc_gather(rtab, rel_idx, R_CH)

    eps = jax.random.normal(jax.random.key(42), (T, D), jnp.float32)
    return _dense(e_rows, r_rows, masks, W_agg, b_agg, W1, b1, W2,
                  b2, W3, b3, Wmu, bmu, Wsig, bsig, Wd, bd, eps)
